# Initial kernel scaffold; baseline (speedup 1.0000x reference)
import jax
import jax.numpy as jnp
from jax.experimental import pallas as pl


def kernel(x, edge_index, W1, b1, W2, b2):
    raise NotImplementedError("write your pallas kernel here")


# speedup vs baseline: 10.6851x; 10.6851x over previous
import functools

import jax
import jax.numpy as jnp
from jax import lax
from jax.experimental import pallas as pl
from jax.experimental.pallas import tpu as pltpu
from jax.experimental.pallas import tpu_sc as plsc

NC = 2
NS = 16
K = 128
BR = 256


def _make_deg_kernel(NPAD, C, interpret=False):
  rpt = NPAD // NS

  @functools.partial(
      pl.kernel,
      out_type=jax.ShapeDtypeStruct((NC, NPAD, 8), jnp.float32),
      mesh=plsc.VectorSubcoreMesh(core_axis_name="c", subcore_axis_name="s",
                                  num_cores=NC, num_subcores=NS),
      scratch_types=[
          pltpu.VMEM_SHARED((NPAD, 8), jnp.float32),
          pltpu.VMEM((K,), jnp.int32),
          pltpu.VMEM((K, 8), jnp.float32),
          pltpu.SemaphoreType.DMA,
      ],
      interpret=interpret,
  )
  def deg_kernel(dst_hbm, ones_hbm, zeros_hbm, out_hbm, acc, dbuf, ones_v,
                 sem):
    c = lax.axis_index("c")
    s = lax.axis_index("s")
    pltpu.sync_copy(zeros_hbm, acc.at[pl.ds(s * rpt, rpt)])
    pltpu.sync_copy(ones_hbm, ones_v)
    plsc.subcore_barrier()
    start = (c * NS + s) * C * K

    def step(i, carry):
      off = start + i * K
      pltpu.sync_copy(dst_hbm.at[pl.ds(off, K)], dbuf)
      pltpu.async_copy(ones_v, acc.at[dbuf], sem, add=True).wait()
      return carry

    lax.fori_loop(0, C, step, 0)
    plsc.subcore_barrier()
    pltpu.sync_copy(acc.at[pl.ds(s * rpt, rpt)],
                    out_hbm.at[c, pl.ds(s * rpt, rpt)])

  return deg_kernel


def _make_agg_kernel(NPAD, D, C, interpret=False):
  rpt = NPAD // NS

  @functools.partial(
      pl.kernel,
      out_type=jax.ShapeDtypeStruct((NC, NPAD, D), jnp.float32),
      mesh=plsc.VectorSubcoreMesh(core_axis_name="c", subcore_axis_name="s",
                                  num_cores=NC, num_subcores=NS),
      scratch_types=[
          pltpu.VMEM_SHARED((NPAD, D), jnp.float32),
          pltpu.VMEM((K,), jnp.int32),
          pltpu.VMEM((K,), jnp.int32),
          pltpu.VMEM((K, D), jnp.float32),
          pltpu.SemaphoreType.DMA,
          pltpu.SemaphoreType.DMA,
      ],
      interpret=interpret,
  )
  def agg_kernel(g_hbm, src_hbm, dst_hbm, zeros_hbm, out_hbm, acc, sbuf,
                 dbuf, rows, sem_g, sem_s):
    c = lax.axis_index("c")
    s = lax.axis_index("s")
    pltpu.sync_copy(zeros_hbm, acc.at[pl.ds(s * rpt, rpt)])
    plsc.subcore_barrier()
    start = (c * NS + s) * C * K

    def step(i, carry):
      off = start + i * K
      pltpu.sync_copy(src_hbm.at[pl.ds(off, K)], sbuf)
      pltpu.sync_copy(dst_hbm.at[pl.ds(off, K)], dbuf)
      pltpu.async_copy(g_hbm.at[sbuf], rows, sem_g).wait()
      pltpu.async_copy(rows, acc.at[dbuf], sem_s, add=True).wait()
      return carry

    lax.fori_loop(0, C, step, 0)
    plsc.subcore_barrier()
    pltpu.sync_copy(acc.at[pl.ds(s * rpt, rpt)],
                    out_hbm.at[c, pl.ds(s * rpt, rpt)])

  return agg_kernel


def _mm_body(x_ref, w_ref, o_ref):
  o_ref[...] = jnp.dot(x_ref[...], w_ref[...],
                       preferred_element_type=jnp.float32)


def _scale_body(h_ref, deg_ref, g_ref, dinv_ref):
  deg = deg_ref[0] + deg_ref[1] + 1.0
  dinv = lax.rsqrt(deg)
  dinv_ref[...] = dinv
  g_ref[...] = h_ref[...] * dinv[:, :1]


def _layer_body(p_ref, g_ref, dinv_ref, b_ref, w_ref, o_ref):
  dinv = dinv_ref[:, :1]
  t = dinv * (p_ref[0] + p_ref[1] + g_ref[...]) + b_ref[...]
  t = jnp.maximum(t, 0.0)
  o_ref[...] = dinv * jnp.dot(t, w_ref[...],
                              preferred_element_type=jnp.float32)


def _final_body(q_ref, g_ref, dinv_ref, b_ref, o_ref):
  dinv = dinv_ref[:, :1]
  t = dinv * (q_ref[0] + q_ref[1] + g_ref[...]) + b_ref[...]
  o_ref[...] = jnp.maximum(t, 0.0)


def _gcn(x, edge_index, W1, b1, W2, b2, interpret=False):
  N, D = x.shape
  E = edge_index.shape[1]
  NPAD = ((N + 2047) // 2048) * 2048
  C = -(-E // (NC * NS * K))
  EPAD = NC * NS * K * C

  xp = jnp.zeros((NPAD, D), jnp.float32).at[:N].set(x)
  pad = jnp.full((EPAD - E,), N, jnp.int32)
  srcp = jnp.concatenate([edge_index[0], pad])
  dstp = jnp.concatenate([edge_index[1], pad])

  ones8 = jnp.ones((K, 8), jnp.float32)
  zeros8 = jnp.zeros((NPAD // NS, 8), jnp.float32)
  zerosD = jnp.zeros((NPAD // NS, D), jnp.float32)

  deg8 = _make_deg_kernel(NPAD, C, interpret)(dstp, ones8, zeros8)

  grid = (NPAD // BR,)
  blk = pl.BlockSpec((BR, D), lambda i: (i, 0))
  blk8 = pl.BlockSpec((BR, 8), lambda i: (i, 0))
  blk2 = pl.BlockSpec((NC, BR, D), lambda i: (0, i, 0))
  blk28 = pl.BlockSpec((NC, BR, 8), lambda i: (0, i, 0))
  blkw = pl.BlockSpec((D, D), lambda i: (0, 0))
  blkb = pl.BlockSpec((1, D), lambda i: (0, 0))
  fD = jax.ShapeDtypeStruct((NPAD, D), jnp.float32)
  f8 = jax.ShapeDtypeStruct((NPAD, 8), jnp.float32)

  h1 = pl.pallas_call(
      _mm_body, grid=grid, in_specs=[blk, blkw], out_specs=blk,
      out_shape=fD, interpret=interpret)(xp, W1)

  g1, dinv8 = pl.pallas_call(
      _scale_body, grid=grid, in_specs=[blk, blk28],
      out_specs=[blk, blk8], out_shape=[fD, f8],
      interpret=interpret)(h1, deg8)

  agg = _make_agg_kernel(NPAD, D, C, interpret)
  P = agg(g1, srcp, dstp, zerosD)

  b1r = b1.reshape(1, D)
  b2r = b2.reshape(1, D)
  g2 = pl.pallas_call(
      _layer_body, grid=grid, in_specs=[blk2, blk, blk8, blkb, blkw],
      out_specs=blk, out_shape=fD, interpret=interpret)(
          P, g1, dinv8, b1r, W2)

  Q = agg(g2, srcp, dstp, zerosD)

  out = pl.pallas_call(
      _final_body, grid=grid, in_specs=[blk2, blk, blk8, blkb],
      out_specs=blk, out_shape=fD, interpret=interpret)(
          Q, g2, dinv8, b2r)

  return out[:N]


def kernel(x, edge_index, W1, b1, W2, b2):
  return _gcn(x, edge_index, W1, b1, W2, b2)
